# baseline (speedup 1.0000x reference)
import functools
import math

import jax
import jax.numpy as jnp
from jax import lax
from jax.experimental import pallas as pl
from jax.experimental.pallas import tpu as pltpu
from jax.experimental.pallas import tpu_sc as plsc

_TEMP_INV = 1.0 / math.e
_NE = 8
_NA = 5


def _scores_body(x_ref, w_ref, b_ref, o_ref):
    s = lax.dot_general(
        w_ref[...], x_ref[...], (((1,), (1,)), ((), ())),
        preferred_element_type=jnp.float32)
    o_ref[...] = (s + b_ref[...]) * _TEMP_INV


def _scores_tc(x, W, b):
    n, d = x.shape
    rows = 4096
    return pl.pallas_call(
        _scores_body,
        grid=(n // rows,),
        in_specs=[
            pl.BlockSpec((rows, d), lambda i: (i, 0)),
            pl.BlockSpec((_NE, d), lambda i: (0, 0)),
            pl.BlockSpec((_NE, 1), lambda i: (0, 0)),
        ],
        out_specs=pl.BlockSpec((_NE, rows), lambda i: (0, i)),
        out_shape=jax.ShapeDtypeStruct((_NE, n), jnp.float32),
    )(x, W, b.reshape(_NE, 1))


def _gate_sc(s_t):
    ne, n = s_t.shape
    info = plsc.get_sparse_core_info()
    nw = info.num_cores * info.num_subcores
    lanes = info.num_lanes
    chunk = n // nw
    mesh = plsc.VectorSubcoreMesh(core_axis_name="c", subcore_axis_name="s")

    @functools.partial(
        pl.kernel,
        out_type=jax.ShapeDtypeStruct((ne, n), jnp.float32),
        mesh=mesh,
        scratch_types=[
            pltpu.VMEM((ne, chunk), jnp.float32),
            pltpu.VMEM((ne, chunk), jnp.float32),
        ],
    )
    def gate(s_hbm, o_hbm, s_v, o_v):
        wid = lax.axis_index("s") * info.num_cores + lax.axis_index("c")
        base = wid * chunk
        pltpu.sync_copy(s_hbm.at[:, pl.ds(base, chunk)], s_v)

        def step(g, carry):
            col = g * lanes
            sv = [s_v[e, pl.ds(col, lanes)] for e in range(_NE)]
            m = sv[0]
            for e in range(1, _NE):
                m = jnp.maximum(m, sv[e])
            ev = [jnp.exp(v - m) for v in sv]
            z = ev[0]
            for e in range(1, _NE):
                z = z + ev[e]
            pv = [v / z for v in ev]

            one = jnp.ones((lanes,), jnp.float32)
            zero = jnp.zeros((lanes,), jnp.float32)
            kept = []
            for i in range(_NE):
                rank = zero
                for j in range(_NE):
                    if j < i:
                        hit = pv[j] >= pv[i]
                    elif j > i:
                        hit = pv[j] > pv[i]
                    else:
                        continue
                    rank = rank + jnp.where(hit, one, zero)
                keep = rank < (_NA - 0.5)
                kept.append(jnp.where(keep, pv[i], zero))
            denom = kept[0]
            for e in range(1, _NE):
                denom = denom + kept[e]
            denom = denom + 1e-8
            for e in range(_NE):
                o_v[e, pl.ds(col, lanes)] = kept[e] / denom
            return carry

        lax.fori_loop(0, chunk // lanes, step, 0)
        pltpu.sync_copy(o_v, o_hbm.at[:, pl.ds(base, chunk)])

    return gate(s_t)


def kernel(x, W, b):
    s_t = _scores_tc(x, W, b)
    o_t = _gate_sc(s_t)
    return o_t.T


# speedup vs baseline: 1.2056x; 1.2056x over previous
import math

import jax
import jax.numpy as jnp
from jax.experimental import pallas as pl

_TEMP_INV = 1.0 / math.e
_N_EXPERTS = 8
_N_ACTIVE = 5


def _gate_body(x_ref, w_ref, b_ref, o_ref):
    x = x_ref[...]
    w = w_ref[...]
    s = jax.lax.dot_general(
        w, x, (((1,), (1,)), ((), ())),
        preferred_element_type=jnp.float32)
    s = (s + b_ref[...]) * _TEMP_INV
    m = jnp.max(s, axis=0, keepdims=True)
    e = jnp.exp(s - m)
    z = jnp.sum(e, axis=0, keepdims=True)
    p = e / z

    rows = []
    for i in range(_N_EXPERTS):
        pi = p[i:i + 1, :]
        gt = (p > pi).astype(jnp.float32)
        tie = (p[:i] == pi).astype(jnp.float32) if i else None
        rank = jnp.sum(gt, axis=0, keepdims=True)
        if tie is not None:
            rank = rank + jnp.sum(tie, axis=0, keepdims=True)
        rows.append((rank < _N_ACTIVE).astype(jnp.float32))
    keep = jnp.concatenate(rows, axis=0)

    kept = p * keep
    denom = jnp.sum(kept, axis=0, keepdims=True) + 1e-8
    o_ref[...] = (kept / denom).T


def kernel(x, W, b):
    n, d = x.shape
    rows = 4096
    grid = (n // rows,)
    b2 = b.reshape(_N_EXPERTS, 1)
    out_t = pl.pallas_call(
        _gate_body,
        grid=grid,
        in_specs=[
            pl.BlockSpec((rows, d), lambda i: (i, 0)),
            pl.BlockSpec((_N_EXPERTS, d), lambda i: (0, 0)),
            pl.BlockSpec((_N_EXPERTS, 1), lambda i: (0, 0)),
        ],
        out_specs=pl.BlockSpec((rows, _N_EXPERTS), lambda i: (i, 0)),
        out_shape=jax.ShapeDtypeStruct((n, _N_EXPERTS), jnp.float32),
    )(x, W, b2)
    return out_t
